# Initial kernel scaffold; baseline (speedup 1.0000x reference)
import functools

import jax
import jax.numpy as jnp
from jax import lax
from jax.experimental import pallas as pl
from jax.experimental.pallas import tpu as pltpu
from jax.experimental.pallas import tpu_sc as plsc

NC = 2
NS = 16
NW = NC * NS


def _gather_call(B, D, CH):
    b_per_w = B // NW
    n_ch = b_per_w // CH
    mesh = plsc.VectorSubcoreMesh(core_axis_name="c", subcore_axis_name="s",
                                  num_cores=NC)

    @functools.partial(
        pl.kernel,
        out_type=jax.ShapeDtypeStruct((B, D), jnp.float32),
        mesh=mesh,
        scratch_types=[
            pltpu.VMEM((CH,), jnp.int32),
            pltpu.VMEM((CH, D), jnp.float32),
            pltpu.SemaphoreType.DMA,
        ],
    )
    def gather_kernel(table_hbm, idx_hbm, out_hbm, idx_v, rows_v, sem):
        wid = lax.axis_index("s") * NC + lax.axis_index("c")
        base = wid * b_per_w

        def body(i, carry):
            start = base + i * CH
            pltpu.sync_copy(idx_hbm.at[pl.ds(start, CH)], idx_v)
            pltpu.async_copy(table_hbm.at[idx_v], rows_v, sem).wait()
            pltpu.sync_copy(rows_v, out_hbm.at[pl.ds(start, CH)])
            return carry

        lax.fori_loop(0, n_ch, body, 0)

    return gather_kernel


def kernel(weight, nodes):
    D = weight.shape[1]
    B = nodes.size
    idx = nodes.reshape(B)
    out = _gather_call(B, D, CH=1024)(weight, idx)
    return out.reshape(nodes.shape + (D,))


# speedup vs baseline: 1.0941x; 1.0941x over previous
import functools

import jax
import jax.numpy as jnp
from jax import lax
from jax.experimental import pallas as pl
from jax.experimental.pallas import tpu as pltpu
from jax.experimental.pallas import tpu_sc as plsc

NC = 2
NS = 16
NW = NC * NS


def _gather_call(B, D, CH):
    b_per_w = B // NW
    n_ch = b_per_w // CH
    mesh = plsc.VectorSubcoreMesh(core_axis_name="c", subcore_axis_name="s",
                                  num_cores=NC)

    @functools.partial(
        pl.kernel,
        out_type=jax.ShapeDtypeStruct((B, D), jnp.float32),
        mesh=mesh,
        scratch_types=[
            pltpu.VMEM((CH,), jnp.int32),
            pltpu.VMEM((CH, D), jnp.float32),
            pltpu.SemaphoreType.DMA,
        ],
        compiler_params=pltpu.CompilerParams(use_tc_tiling_on_sc=False),
    )
    def gather_kernel(table_hbm, idx_hbm, out_hbm, idx_v, rows_v, sem):
        wid = lax.axis_index("s") * NC + lax.axis_index("c")
        base = wid * b_per_w

        def body(i, carry):
            start = base + i * CH
            pltpu.sync_copy(idx_hbm.at[pl.ds(start, CH)], idx_v)
            pltpu.async_copy(table_hbm.at[idx_v], rows_v, sem).wait()
            pltpu.sync_copy(rows_v, out_hbm.at[pl.ds(start, CH)])
            return carry

        lax.fori_loop(0, n_ch, body, 0)

    return gather_kernel


def kernel(weight, nodes):
    D = weight.shape[1]
    B = nodes.size
    idx = nodes.reshape(B)
    out = _gather_call(B, D, CH=1024)(weight, idx)
    return out.reshape(nodes.shape + (D,))


# speedup vs baseline: 1.1130x; 1.0173x over previous
import functools

import jax
import jax.numpy as jnp
from jax import lax
from jax.experimental import pallas as pl
from jax.experimental.pallas import tpu as pltpu
from jax.experimental.pallas import tpu_sc as plsc

NC = 2
NS = 16
NW = NC * NS


def _gather_call(B, D, CH, NBUF, GD):
    b_per_w = B // NW
    n_ch = b_per_w // CH
    assert b_per_w % CH == 0 and n_ch % NBUF == 0
    n_outer = n_ch // NBUF
    assert n_outer >= 2
    WD = NBUF - GD
    assert 1 <= GD < NBUF and WD >= 1

    mesh = plsc.VectorSubcoreMesh(core_axis_name="c", subcore_axis_name="s",
                                  num_cores=NC)
    scratch = (
        [pltpu.VMEM((CH,), jnp.int32) for _ in range(NBUF)]
        + [pltpu.VMEM((CH, D), jnp.float32) for _ in range(NBUF)]
        + [pltpu.SemaphoreType.DMA for _ in range(2 * NBUF)]
    )

    @functools.partial(
        pl.kernel,
        out_type=jax.ShapeDtypeStruct((B, D), jnp.float32),
        mesh=mesh,
        scratch_types=scratch,
        compiler_params=pltpu.CompilerParams(use_tc_tiling_on_sc=False),
    )
    def gather_kernel(table_hbm, idx_hbm, out_hbm, *refs):
        idx_v = refs[0:NBUF]
        rows_v = refs[NBUF:2 * NBUF]
        sem_g = refs[2 * NBUF:3 * NBUF]
        sem_w = refs[3 * NBUF:4 * NBUF]
        wid = lax.axis_index("s") * NC + lax.axis_index("c")
        base = wid * b_per_w

        def start_gather(c, b):
            pltpu.sync_copy(idx_hbm.at[pl.ds(base + c * CH, CH)], idx_v[b])
            pltpu.async_copy(table_hbm.at[idx_v[b]], rows_v[b], sem_g[b])

        def wait_gather(b):
            pltpu.make_async_copy(table_hbm.at[idx_v[b]], rows_v[b],
                                  sem_g[b]).wait()

        def start_wb(c, b):
            pltpu.async_copy(rows_v[b], out_hbm.at[pl.ds(base + c * CH, CH)],
                             sem_w[b])

        def wait_wb(b):
            pltpu.make_async_copy(rows_v[b], out_hbm.at[pl.ds(base, CH)],
                                  sem_w[b]).wait()

        def step(r, b, do_input, do_waitwb):
            ib = (b + GD) % NBUF
            if do_input:
                if do_waitwb:
                    wait_wb(ib)
                start_gather(r + GD, ib)
            wait_gather(b)
            start_wb(r, b)

        for c in range(GD):
            start_gather(c, c)
        for b in range(NBUF):
            step(b, b, True, b >= WD)

        def outer(o, carry):
            r0 = o * NBUF
            for b in range(NBUF):
                step(r0 + b, b, True, True)
            return carry

        lax.fori_loop(1, n_outer - 1, outer, 0)

        r0 = (n_outer - 1) * NBUF
        for b in range(NBUF):
            step(r0 + b, b, b < WD, True)
        for b in range(NBUF):
            wait_wb(b)

    return gather_kernel


def kernel(weight, nodes):
    D = weight.shape[1]
    B = nodes.size
    idx = nodes.reshape(B)
    out = _gather_call(B, D, CH=800, NBUF=4, GD=2)(weight, idx)
    return out.reshape(nodes.shape + (D,))


# speedup vs baseline: 1.9415x; 1.7444x over previous
import functools

import jax
import jax.numpy as jnp
from jax import lax
from jax.experimental import pallas as pl
from jax.experimental.pallas import tpu as pltpu
from jax.experimental.pallas import tpu_sc as plsc

NC = 2
NS = 16
NW = NC * NS


def _gather_call(V, D, J, B, NBUF, GD):
    CH = B // NW
    n_ch = J
    WD = NBUF - GD
    assert 1 <= GD < NBUF and n_ch % NBUF == 0
    n_outer = n_ch // NBUF
    assert n_outer >= 2

    mesh = plsc.VectorSubcoreMesh(core_axis_name="c", subcore_axis_name="s",
                                  num_cores=NC)
    scratch = (
        [pltpu.VMEM((CH,), jnp.int32) for _ in range(NBUF)]
        + [pltpu.VMEM((CH, D), jnp.float32) for _ in range(NBUF)]
        + [pltpu.SemaphoreType.DMA for _ in range(2 * NBUF)]
    )

    @functools.partial(
        pl.kernel,
        out_type=jax.ShapeDtypeStruct((J, B, D), jnp.float32),
        mesh=mesh,
        scratch_types=scratch,
        compiler_params=pltpu.CompilerParams(use_tc_tiling_on_sc=False),
    )
    def gather_kernel(table_hbm, idx_hbm, out_hbm, *refs):
        idx_v = refs[0:NBUF]
        rows_v = refs[NBUF:2 * NBUF]
        sem_g = refs[2 * NBUF:3 * NBUF]
        sem_w = refs[3 * NBUF:4 * NBUF]
        wid = lax.axis_index("s") * NC + lax.axis_index("c")
        boff = wid * CH

        def start_gather(c, b):
            pltpu.sync_copy(idx_hbm.at[c, pl.ds(boff, CH)], idx_v[b])
            pltpu.async_copy(table_hbm.at[idx_v[b]], rows_v[b], sem_g[b])

        def wait_gather(b):
            pltpu.make_async_copy(table_hbm.at[idx_v[b]], rows_v[b],
                                  sem_g[b]).wait()

        def start_wb(c, b):
            pltpu.async_copy(rows_v[b], out_hbm.at[c, pl.ds(boff, CH)],
                             sem_w[b])

        def wait_wb(b):
            pltpu.make_async_copy(rows_v[b], out_hbm.at[0, pl.ds(boff, CH)],
                                  sem_w[b]).wait()

        def step(r, b, do_input, do_waitwb):
            ib = (b + GD) % NBUF
            if do_input:
                if do_waitwb:
                    wait_wb(ib)
                start_gather(r + GD, ib)
            wait_gather(b)
            start_wb(r, b)

        for c in range(GD):
            start_gather(c, c)
        for b in range(NBUF):
            step(b, b, True, b >= WD)

        def outer(o, carry):
            r0 = o * NBUF
            for b in range(NBUF):
                step(r0 + b, b, True, True)
            return carry

        lax.fori_loop(1, n_outer - 1, outer, 0)

        r0 = (n_outer - 1) * NBUF
        for b in range(NBUF):
            step(r0 + b, b, b < WD, True)
        for b in range(NBUF):
            wait_wb(b)

    return gather_kernel


def kernel(weight, nodes):
    V, D = weight.shape
    Bt, J = nodes.shape
    nodes_t = jnp.transpose(nodes)
    out_t = _gather_call(V, D, J, Bt, NBUF=5, GD=2)(weight, nodes_t)
    return jnp.transpose(out_t, (1, 0, 2))


# speedup vs baseline: 1.9427x; 1.0006x over previous
import functools

import jax
import jax.numpy as jnp
from jax import lax
from jax.experimental import pallas as pl
from jax.experimental.pallas import tpu as pltpu
from jax.experimental.pallas import tpu_sc as plsc

NC = 2
NS = 16
NW = NC * NS


def _gather_call(V, D, J, B, NBUF, GD):
    CH = B // NW
    n_ch = J
    WD = NBUF - GD
    assert 1 <= GD < NBUF and n_ch % NBUF == 0
    n_outer = n_ch // NBUF
    assert n_outer >= 2

    mesh = plsc.VectorSubcoreMesh(core_axis_name="c", subcore_axis_name="s",
                                  num_cores=NC)
    scratch = (
        [pltpu.VMEM((CH,), jnp.int32) for _ in range(NBUF)]
        + [pltpu.VMEM((CH, D), jnp.float32) for _ in range(NBUF)]
        + [pltpu.SemaphoreType.DMA for _ in range(2 * NBUF)]
    )

    @functools.partial(
        pl.kernel,
        out_type=jax.ShapeDtypeStruct((J * B, D), jnp.float32),
        mesh=mesh,
        scratch_types=scratch,
        compiler_params=pltpu.CompilerParams(use_tc_tiling_on_sc=False),
    )
    def gather_kernel(table_hbm, idx_hbm, out_hbm, *refs):
        idx_v = refs[0:NBUF]
        rows_v = refs[NBUF:2 * NBUF]
        sem_g = refs[2 * NBUF:3 * NBUF]
        sem_w = refs[3 * NBUF:4 * NBUF]
        wid = lax.axis_index("s") * NC + lax.axis_index("c")
        boff = wid * CH

        def start_gather(c, b):
            pltpu.sync_copy(idx_hbm.at[c, pl.ds(boff, CH)], idx_v[b])
            pltpu.async_copy(table_hbm.at[idx_v[b]], rows_v[b], sem_g[b])

        def wait_gather(b):
            pltpu.make_async_copy(table_hbm.at[idx_v[b]], rows_v[b],
                                  sem_g[b]).wait()

        def start_wb(c, b):
            pltpu.async_copy(rows_v[b], out_hbm.at[pl.ds(c * B + boff, CH)],
                             sem_w[b])

        def wait_wb(b):
            pltpu.make_async_copy(rows_v[b], out_hbm.at[pl.ds(boff, CH)],
                                  sem_w[b]).wait()

        def step(r, b, do_input, do_waitwb):
            ib = (b + GD) % NBUF
            if do_input:
                if do_waitwb:
                    wait_wb(ib)
                start_gather(r + GD, ib)
            wait_gather(b)
            start_wb(r, b)

        for c in range(GD):
            start_gather(c, c)
        for b in range(NBUF):
            step(b, b, True, b >= WD)

        def outer(o, carry):
            r0 = o * NBUF
            for b in range(NBUF):
                step(r0 + b, b, True, True)
            return carry

        lax.fori_loop(1, n_outer - 1, outer, 0)

        r0 = (n_outer - 1) * NBUF
        for b in range(NBUF):
            step(r0 + b, b, b < WD, True)
        for b in range(NBUF):
            wait_wb(b)

    return gather_kernel


def kernel(weight, nodes):
    V, D = weight.shape
    Bt, J = nodes.shape
    nodes_t = jnp.transpose(nodes)
    out2d = _gather_call(V, D, J, Bt, NBUF=5, GD=2)(weight, nodes_t)
    out_t = out2d.reshape(J, Bt, D)
    return jnp.transpose(out_t, (1, 0, 2))
